# baseline (speedup 1.0000x reference)
import functools

import jax
import jax.numpy as jnp
from jax import lax
from jax.experimental import pallas as pl
from jax.experimental.pallas import tpu as pltpu
from jax.experimental.pallas import tpu_sc as plsc

_N_ROWS = 1000000
_DIM = 64
_NUM_WORKERS = 32
_ROWS_PER_WORKER = 31248
_CHUNK_ROWS = 496
_CHUNKS_PER_TABLE = _ROWS_PER_WORKER // _CHUNK_ROWS
_TAIL_BASE = _NUM_WORKERS * _ROWS_PER_WORKER
_TAIL_ROWS = _N_ROWS - _TAIL_BASE
_N_BUF = 2

_mesh = plsc.VectorSubcoreMesh(core_axis_name="c", subcore_axis_name="s")


@functools.partial(
    pl.kernel,
    out_type=(
        jax.ShapeDtypeStruct((_N_ROWS, _DIM), jnp.float32),
        jax.ShapeDtypeStruct((_N_ROWS, _DIM), jnp.float32),
    ),
    mesh=_mesh,
    scratch_types=[
        pltpu.VMEM((_CHUNK_ROWS, _DIM), jnp.float32),
        pltpu.VMEM((_CHUNK_ROWS, _DIM), jnp.float32),
        pltpu.SemaphoreType.DMA,
        pltpu.SemaphoreType.DMA,
        pltpu.SemaphoreType.DMA,
        pltpu.SemaphoreType.DMA,
    ],
)
def _sc_copy(u_hbm, i_hbm, out_u, out_i, b0, b1, si0, si1, so0, so1):
    bufs = (b0, b1)
    in_sems = (si0, si1)
    out_sems = (so0, so1)
    wid = lax.axis_index("c") * 16 + lax.axis_index("s")
    base = wid * _ROWS_PER_WORKER

    tasks = []
    for k in range(_CHUNKS_PER_TABLE):
        tasks.append((u_hbm, out_u, k))
        tasks.append((i_hbm, out_i, k))

    def in_copy(t):
        src, _, k = tasks[t]
        slot = t % _N_BUF
        return pltpu.make_async_copy(
            src.at[pl.ds(base + k * _CHUNK_ROWS, _CHUNK_ROWS), :],
            bufs[slot],
            in_sems[slot],
        )

    def out_copy(t):
        _, dst, k = tasks[t]
        slot = t % _N_BUF
        return pltpu.make_async_copy(
            bufs[slot],
            dst.at[pl.ds(base + k * _CHUNK_ROWS, _CHUNK_ROWS), :],
            out_sems[slot],
        )

    T = len(tasks)
    for t in range(min(_N_BUF, T)):
        in_copy(t).start()
    for t in range(T):
        in_copy(t).wait()
        out_copy(t).start()
        nt = t + _N_BUF
        if nt < T:
            out_copy(t).wait()
            in_copy(nt).start()
    for t in range(max(T - _N_BUF, 0), T):
        out_copy(t).wait()

    @pl.when(wid == 0)
    def _():
        for src, dst, slot in ((u_hbm, out_u, 0), (i_hbm, out_i, 1)):
            pltpu.make_async_copy(
                src.at[pl.ds(_TAIL_BASE, _TAIL_ROWS), :],
                bufs[slot].at[pl.ds(0, _TAIL_ROWS), :],
                in_sems[slot],
            ).start()
        for src, dst, slot in ((u_hbm, out_u, 0), (i_hbm, out_i, 1)):
            pltpu.make_async_copy(
                src.at[pl.ds(_TAIL_BASE, _TAIL_ROWS), :],
                bufs[slot].at[pl.ds(0, _TAIL_ROWS), :],
                in_sems[slot],
            ).wait()
            pltpu.make_async_copy(
                bufs[slot].at[pl.ds(0, _TAIL_ROWS), :],
                dst.at[pl.ds(_TAIL_BASE, _TAIL_ROWS), :],
                out_sems[slot],
            ).start()
        for src, dst, slot in ((u_hbm, out_u, 0), (i_hbm, out_i, 1)):
            pltpu.make_async_copy(
                bufs[slot].at[pl.ds(0, _TAIL_ROWS), :],
                dst.at[pl.ds(_TAIL_BASE, _TAIL_ROWS), :],
                out_sems[slot],
            ).wait()


def kernel(user_history, user_table, item_table):
    del user_history
    user_emb, item_emb = _sc_copy(user_table, item_table)
    return (user_emb, item_emb)
